# baseline (speedup 1.0000x reference)
import jax
import jax.numpy as jnp
from jax import lax
from jax.experimental import pallas as pl
from jax.experimental.pallas import tpu as pltpu
from jax.experimental.pallas import tpu_sc as plsc

H = 128
D_E = 16
N = 10000
E = 320000
L = 50000
LP = 51200
NPAD = 10240
SR = NPAD // 16
B = 80
BP = 80
EPT2 = E // 16
STEPS2 = EPT2 // B
ZR = 80
PADC = 128


def _proj_body(x_ref, w_ref, b_ref, k_ref, s_ref, q_ref, v_ref):
    acc = jnp.dot(x_ref[0], w_ref[0], preferred_element_type=jnp.float32)
    acc = acc + b_ref[0]
    k_ref[...] = acc[:, 0:H]
    s_ref[...] = acc[:, H:2 * H]
    q_ref[...] = acc[:, 2 * H:3 * H]
    v_ref[...] = acc[:, 3 * H:4 * H]


def _proj(x_st, w_st, b_st, rows=1000):
    nb = N // rows
    return pl.pallas_call(
        _proj_body,
        grid=(2, nb),
        in_specs=[
            pl.BlockSpec((1, rows, H), lambda j, i: (j, i, 0)),
            pl.BlockSpec((1, H, 4 * H), lambda j, i: (j, 0, 0)),
            pl.BlockSpec((1, 1, 4 * H), lambda j, i: (j, 0, 0)),
        ],
        out_specs=[
            pl.BlockSpec((rows, H), lambda j, i, nb=nb: (j * nb + i, 0)),
            pl.BlockSpec((rows, H), lambda j, i, nb=nb: (j * nb + i, 0)),
            pl.BlockSpec((rows, H), lambda j, i, nb=nb: ((1 - j) * nb + i, 0)),
            pl.BlockSpec((rows, H), lambda j, i, nb=nb: ((1 - j) * nb + i, 0)),
        ],
        out_shape=[jax.ShapeDtypeStruct((2 * N, H), jnp.float32)
                   for _ in range(4)],
    )(x_st, w_st, b_st)


def _combine_body(a_ref, s_ref, w_ref, b_ref, k_ref, s2_ref, q_ref, v_ref):
    h = jnp.maximum(a_ref[...] + s_ref[...], 0.0)
    acc = jnp.dot(h, w_ref[0], preferred_element_type=jnp.float32)
    acc = acc + b_ref[0]
    k_ref[...] = acc[:, 0:H]
    s2_ref[...] = acc[:, H:2 * H]
    q_ref[...] = acc[:, 2 * H:3 * H]
    v_ref[...] = acc[:, 3 * H:4 * H]


def _combine(agg_st, s_st, w_st, b_st, rows=80):
    nb = N // rows
    nbp = NPAD // rows
    return pl.pallas_call(
        _combine_body,
        grid=(2, nb),
        in_specs=[
            pl.BlockSpec((rows, H), lambda j, i, nbp=nbp: (j * nbp + i, 0)),
            pl.BlockSpec((rows, H), lambda j, i, nb=nb: (j * nb + i, 0)),
            pl.BlockSpec((1, H, 4 * H), lambda j, i: (j, 0, 0)),
            pl.BlockSpec((1, 1, 4 * H), lambda j, i: (j, 0, 0)),
        ],
        out_specs=[
            pl.BlockSpec((rows, H), lambda j, i, nb=nb: (j * nb + i, 0)),
            pl.BlockSpec((rows, H), lambda j, i, nb=nb: (j * nb + i, 0)),
            pl.BlockSpec((rows, H), lambda j, i, nb=nb: ((1 - j) * nb + i, 0)),
            pl.BlockSpec((rows, H), lambda j, i, nb=nb: ((1 - j) * nb + i, 0)),
        ],
        out_shape=[jax.ShapeDtypeStruct((2 * N, H), jnp.float32)
                   for _ in range(4)],
    )(agg_st, s_st, w_st, b_st)


def _cls_body(a_ref, s_ref, w_ref, b_ref, y_ref):
    h = jnp.maximum(a_ref[...] + s_ref[...], 0.0)
    y_ref[...] = jnp.dot(h, w_ref[0],
                         preferred_element_type=jnp.float32) + b_ref[0]


def _cls(agg_st, s_st, w_st, b_st, rows=80):
    nb = N // rows
    nbp = NPAD // rows
    return pl.pallas_call(
        _cls_body,
        grid=(2, nb),
        in_specs=[
            pl.BlockSpec((rows, H), lambda j, i, nbp=nbp: (j * nbp + i, 0)),
            pl.BlockSpec((rows, H), lambda j, i, nb=nb: (j * nb + i, 0)),
            pl.BlockSpec((1, H, PADC), lambda j, i: (j, 0, 0)),
            pl.BlockSpec((1, 1, PADC), lambda j, i: (j, 0, 0)),
        ],
        out_specs=pl.BlockSpec((rows, PADC),
                               lambda j, i, nb=nb: (j * nb + i, 0)),
        out_shape=jax.ShapeDtypeStruct((2 * N, PADC), jnp.float32),
    )(agg_st, s_st, w_st, b_st)


def _eproj_body(x_ref, w_ref, b_ref, o_ref):
    o_ref[...] = (jnp.dot(x_ref[0], w_ref[0],
                          preferred_element_type=jnp.float32) + b_ref[0])


def _eproj(ea_st, w_st, b_st, rows=4000):
    nb = E // rows
    return pl.pallas_call(
        _eproj_body,
        grid=(2, nb),
        in_specs=[
            pl.BlockSpec((1, rows, D_E), lambda j, i: (j, i, 0)),
            pl.BlockSpec((1, D_E, H), lambda j, i: (j, 0, 0)),
            pl.BlockSpec((1, 1, H), lambda j, i: (j, 0, 0)),
        ],
        out_specs=pl.BlockSpec((rows, H), lambda j, i, nb=nb: (j * nb + i, 0)),
        out_shape=jax.ShapeDtypeStruct((2 * E, H), jnp.float32),
    )(ea_st, w_st, b_st)


def _sc_edge_body(k_hbm, q_hbm, v_hbm, e_hbm, src_hbm, dst_hbm, out_hbm,
                  src_v, dst_v, dst2_v, kr, qr, vr, er, agg_sh, sem):
    c = lax.axis_index("c")
    s = lax.axis_index("s")

    def zrow(i, carry):
        for j in range(H // 16):
            kr[i, pl.ds(j * 16, 16)] = jnp.zeros((16,), jnp.float32)
        return carry
    lax.fori_loop(0, ZR, zrow, 0)
    for m in range(SR // ZR):
        off = pl.multiple_of(s * SR + m * ZR, 8)
        pltpu.sync_copy(kr, agg_sh.at[pl.ds(off, ZR)])
    plsc.subcore_barrier()

    base0 = c * E + s * EPT2
    roff = c * N

    def step(g, carry):
        base = pl.multiple_of(base0 + g * B, 8)
        pltpu.sync_copy(src_hbm.at[pl.ds(base, B)], src_v)
        pltpu.sync_copy(dst_hbm.at[pl.ds(base, B)], dst_v)
        for t in range(B // 16):
            sl = pl.ds(t * 16, 16)
            src_v[sl] = src_v[sl] + roff
            dst2_v[sl] = dst_v[sl] + roff
        h1 = pltpu.async_copy(k_hbm.at[dst2_v], kr, sem)
        h2 = pltpu.async_copy(q_hbm.at[src_v], qr, sem)
        h3 = pltpu.async_copy(v_hbm.at[src_v], vr, sem)
        h4 = pltpu.async_copy(e_hbm.at[pl.ds(base, B)], er, sem)
        h1.wait()
        h2.wait()
        h3.wait()
        h4.wait()

        def row(i, carry2):
            for j in range(H // 16):
                sl = pl.ds(j * 16, 16)
                x = kr[i, sl] + qr[i, sl] + er[i, sl]
                gate = 1.0 / (1.0 + jnp.exp(-x))
                kr[i, sl] = gate * vr[i, sl]
            return carry2
        lax.fori_loop(0, B, row, 0)
        pltpu.sync_copy(kr, agg_sh.at[dst_v], add=True)
        return carry
    lax.fori_loop(0, STEPS2, step, 0)
    plsc.subcore_barrier()

    for m in range(SR // ZR):
        off = pl.multiple_of(s * SR + m * ZR, 8)
        pltpu.sync_copy(agg_sh.at[pl.ds(off, ZR)],
                        out_hbm.at[pl.ds(pl.multiple_of(c * NPAD + off, 8),
                                         ZR)])


def _sc_edge(k_st, q_st, v_st, e_st, src_st, dst_st):
    mesh = plsc.VectorSubcoreMesh(core_axis_name="c", subcore_axis_name="s")
    f = pl.kernel(
        _sc_edge_body,
        out_type=jax.ShapeDtypeStruct((2 * NPAD, H), jnp.float32),
        mesh=mesh,
        scratch_types=[
            pltpu.VMEM((B,), jnp.int32),
            pltpu.VMEM((B,), jnp.int32),
            pltpu.VMEM((B,), jnp.int32),
            pltpu.VMEM((B, H), jnp.float32),
            pltpu.VMEM((B, H), jnp.float32),
            pltpu.VMEM((B, H), jnp.float32),
            pltpu.VMEM((B, H), jnp.float32),
            pltpu.VMEM_SHARED((NPAD, H), jnp.float32),
            pltpu.SemaphoreType.DMA,
        ],
    )
    return f(k_st, q_st, v_st, e_st, src_st, dst_st)


def _sc_pair_body(y_hbm, i0_hbm, i1_hbm, out_hbm, i0_v, i1_v, mr, vr, sem):
    c = lax.axis_index("c")
    s = lax.axis_index("s")
    base0 = (c * 16 + s) * (LP // 32)

    def step(g, carry):
        base = pl.multiple_of(base0 + g * BP, 8)
        pltpu.sync_copy(i0_hbm.at[pl.ds(base, BP)], i0_v)
        pltpu.sync_copy(i1_hbm.at[pl.ds(base, BP)], i1_v)
        h1 = pltpu.async_copy(y_hbm.at[i0_v], mr, sem)
        h2 = pltpu.async_copy(y_hbm.at[i1_v], vr, sem)
        h1.wait()
        h2.wait()

        def row(i, carry2):
            mr[i, pl.ds(0, 16)] = mr[i, pl.ds(0, 16)] + vr[i, pl.ds(0, 16)]
            return carry2
        lax.fori_loop(0, BP, row, 0)
        pltpu.sync_copy(mr, out_hbm.at[pl.ds(base, BP)])
        return carry
    lax.fori_loop(0, (LP // 32) // BP, step, 0)


def _sc_pairsum(y_st, i0, i1):
    mesh = plsc.VectorSubcoreMesh(core_axis_name="c", subcore_axis_name="s")
    f = pl.kernel(
        _sc_pair_body,
        out_type=jax.ShapeDtypeStruct((LP, PADC), jnp.float32),
        mesh=mesh,
        scratch_types=[
            pltpu.VMEM((BP,), jnp.int32),
            pltpu.VMEM((BP,), jnp.int32),
            pltpu.VMEM((BP, PADC), jnp.float32),
            pltpu.VMEM((BP, PADC), jnp.float32),
            pltpu.SemaphoreType.DMA,
        ],
    )
    return f(y_st, i0, i1)


def kernel(x_mouse, x_virus, edge_attr_mv, edge_attr_vm, params,
           edge_index_mv, edge_index_vm, edge_label_index):
    p = params

    def catw(*names):
        return jnp.concatenate([p[n] for n in names], axis=1)

    def catb(*names):
        return jnp.concatenate([p[n] for n in names])

    wv_cat = catw('l1_mv_Wk', 'l1_mv_Ws', 'l1_vm_Wq', 'l1_vm_Wv')
    bv_cat = catb('l1_mv_bk', 'l1_mv_bs', 'l1_vm_bq', 'l1_vm_bv')
    wm_cat = catw('l1_vm_Wk', 'l1_vm_Ws', 'l1_mv_Wq', 'l1_mv_Wv')
    bm_cat = catb('l1_vm_bk', 'l1_vm_bs', 'l1_mv_bq', 'l1_mv_bv')
    w1_st = jnp.stack([p['virus_lin_W'] @ wv_cat, p['mouse_lin_W'] @ wm_cat])
    b1_st = jnp.stack([(p['virus_lin_b'] @ wv_cat + bv_cat).reshape(1, -1),
                       (p['mouse_lin_b'] @ wm_cat + bm_cat).reshape(1, -1)])
    w2_st = jnp.stack([catw('l2_mv_Wk', 'l2_mv_Ws', 'l2_vm_Wq', 'l2_vm_Wv'),
                       catw('l2_vm_Wk', 'l2_vm_Ws', 'l2_mv_Wq', 'l2_mv_Wv')])
    b2_st = jnp.stack([catb('l2_mv_bk', 'l2_mv_bs', 'l2_vm_bq', 'l2_vm_bv')
                       .reshape(1, -1),
                       catb('l2_vm_bk', 'l2_vm_bs', 'l2_mv_bq', 'l2_mv_bv')
                       .reshape(1, -1)])
    wtop = jnp.zeros((H, PADC), jnp.float32).at[:, :2].set(p['cls_W'][:H])
    wbot = jnp.zeros((H, PADC), jnp.float32).at[:, :2].set(p['cls_W'][H:])
    btop = jnp.zeros((1, PADC), jnp.float32).at[0, :2].set(p['cls_b'])
    wcls_st = jnp.stack([wbot, wtop])
    bcls_st = jnp.stack([jnp.zeros((1, PADC), jnp.float32), btop])
    we1_st = jnp.stack([p['l1_mv_We'], p['l1_vm_We']])
    be1_st = jnp.stack([p['l1_mv_be'].reshape(1, -1),
                        p['l1_vm_be'].reshape(1, -1)])
    we2_st = jnp.stack([p['l2_mv_We'], p['l2_vm_We']])
    be2_st = jnp.stack([p['l2_mv_be'].reshape(1, -1),
                        p['l2_vm_be'].reshape(1, -1)])

    x_st = jnp.stack([x_virus, x_mouse])
    ea_st = jnp.stack([edge_attr_mv, edge_attr_vm])
    src_st = jnp.concatenate([edge_index_mv[0], edge_index_vm[0]])
    dst_st = jnp.concatenate([edge_index_mv[1], edge_index_vm[1]])
    i0 = jnp.pad(edge_label_index[0], (0, LP - L)) + N
    i1 = jnp.pad(edge_label_index[1], (0, LP - L))

    e1_st = _eproj(ea_st, we1_st, be1_st)
    e2_st = _eproj(ea_st, we2_st, be2_st)

    k1, s1, q1, v1 = _proj(x_st, w1_st, b1_st)
    agg1 = _sc_edge(k1, q1, v1, e1_st, src_st, dst_st)
    k2, s2, q2, v2 = _combine(agg1, s1, w2_st, b2_st)
    agg2 = _sc_edge(k2, q2, v2, e2_st, src_st, dst_st)
    y_st = _cls(agg2, s2, wcls_st, bcls_st)
    out = _sc_pairsum(y_st, i0, i1)
    return out[:L, :2]


# speedup vs baseline: 1.2655x; 1.2655x over previous
import functools

import jax
import jax.numpy as jnp
from jax import lax
from jax.experimental import pallas as pl
from jax.experimental.pallas import tpu as pltpu
from jax.experimental.pallas import tpu_sc as plsc

H = 128
D_E = 16
N = 10000
E = 320000
L = 50000
LP = 51200
NTILES = 32
EPT = E // NTILES
B = 40
BP = 80
STEPS = EPT // B
CH = 10
NPAD = 10240
SR = NPAD // 16
ZR = 40
PADC = 128


def _mm_body(x_ref, w_ref, b_ref, *out_refs):
    acc = jnp.dot(x_ref[...], w_ref[...], preferred_element_type=jnp.float32)
    acc = acc + b_ref[...]
    for j, o in enumerate(out_refs):
        o[...] = acc[:, j * H:(j + 1) * H]


def _mm_split(x, w, b, nout, rows):
    n = x.shape[0]
    grid = n // rows
    kdim = x.shape[1]
    return pl.pallas_call(
        _mm_body,
        grid=(grid,),
        in_specs=[
            pl.BlockSpec((rows, kdim), lambda i: (i, 0)),
            pl.BlockSpec((kdim, w.shape[1]), lambda i: (0, 0)),
            pl.BlockSpec((1, w.shape[1]), lambda i: (0, 0)),
        ],
        out_specs=[pl.BlockSpec((rows, H), lambda i: (i, 0))
                   for _ in range(nout)],
        out_shape=[jax.ShapeDtypeStruct((n, H), jnp.float32)
                   for _ in range(nout)],
    )(x, w, b.reshape(1, -1))


def _combine_body(a0_ref, a1_ref, s_ref, w_ref, b_ref, *out_refs):
    h = jnp.maximum(a0_ref[...] + a1_ref[...] + s_ref[...], 0.0)
    acc = jnp.dot(h, w_ref[...], preferred_element_type=jnp.float32)
    acc = acc + b_ref[...]
    wout = w_ref.shape[1] // len(out_refs)
    for j, o in enumerate(out_refs):
        o[...] = acc[:, j * wout:(j + 1) * wout]


def _mm_combine(aggflat, s, w, b, nout, rows=80):
    grid = N // rows
    nblk = NPAD // rows
    wout = w.shape[1] // nout
    return pl.pallas_call(
        _combine_body,
        grid=(grid,),
        in_specs=[
            pl.BlockSpec((rows, H), lambda i: (i, 0)),
            pl.BlockSpec((rows, H), lambda i, nblk=nblk: (i + nblk, 0)),
            pl.BlockSpec((rows, H), lambda i: (i, 0)),
            pl.BlockSpec((H, w.shape[1]), lambda i: (0, 0)),
            pl.BlockSpec((1, w.shape[1]), lambda i: (0, 0)),
        ],
        out_specs=[pl.BlockSpec((rows, wout), lambda i: (i, 0))
                   for _ in range(nout)],
        out_shape=[jax.ShapeDtypeStruct((N, wout), jnp.float32)
                   for _ in range(nout)],
    )(aggflat, aggflat, s, w, b.reshape(1, -1))


def _sc_edge_body(k_hbm, q_hbm, v_hbm, e_hbm, src_hbm, dst_hbm, out_hbm,
                  src0, dst0, kr0, qr0, vr0, er0,
                  src1, dst1, kr1, qr1, vr1, er1, agg_sh, sem0, sem1):
    c = lax.axis_index("c")
    s = lax.axis_index("s")

    def zrow(i, carry):
        for j in range(H // 16):
            kr0[i, pl.ds(j * 16, 16)] = jnp.zeros((16,), jnp.float32)
        return carry
    lax.fori_loop(0, ZR, zrow, 0)
    for m in range(SR // ZR):
        off = pl.multiple_of(s * SR + m * ZR, 8)
        pltpu.sync_copy(kr0, agg_sh.at[pl.ds(off, ZR)])
    plsc.subcore_barrier()

    base0 = (c * 16 + s) * EPT
    sets = ((src0, dst0, kr0, qr0, vr0, er0, sem0),
            (src1, dst1, kr1, qr1, vr1, er1, sem1))

    def issue(g, st):
        sv, dv, kr, qr, vr, er, sem = st
        base = pl.multiple_of(base0 + g * B, 8)
        pltpu.sync_copy(src_hbm.at[pl.ds(base, B)], sv)
        pltpu.sync_copy(dst_hbm.at[pl.ds(base, B)], dv)
        return (pltpu.async_copy(k_hbm.at[dv], kr, sem),
                pltpu.async_copy(q_hbm.at[sv], qr, sem),
                pltpu.async_copy(v_hbm.at[sv], vr, sem),
                pltpu.async_copy(e_hbm.at[pl.ds(base, B)], er, sem))

    def compute(st, hs):
        sv, dv, kr, qr, vr, er, sem = st
        for h in hs:
            h.wait()

        def row(i, carry2):
            for j in range(H // 16):
                sl = pl.ds(j * 16, 16)
                x = kr[i, sl] + qr[i, sl] + er[i, sl]
                gate = 1.0 / (1.0 + jnp.exp(-x))
                kr[i, sl] = gate * vr[i, sl]
            return carry2
        lax.fori_loop(0, B, row, 0)
        pltpu.sync_copy(kr, agg_sh.at[dv], add=True)

    def chunk(cc, carry):
        g0 = cc * CH
        hs = issue(g0, sets[0])
        for u in range(CH):
            nh = issue(g0 + u + 1, sets[(u + 1) % 2]) if u + 1 < CH else None
            compute(sets[u % 2], hs)
            hs = nh
        return carry
    lax.fori_loop(0, STEPS // CH, chunk, 0)
    plsc.subcore_barrier()

    for m in range(SR // ZR):
        off = pl.multiple_of(s * SR + m * ZR, 8)
        pltpu.sync_copy(agg_sh.at[pl.ds(off, ZR)],
                        out_hbm.at[pl.ds(pl.multiple_of(c * NPAD + off, 8),
                                         ZR)])


def _sc_edge(k_tab, q_tab, v_tab, e_tab, src, dst):
    mesh = plsc.VectorSubcoreMesh(core_axis_name="c", subcore_axis_name="s")
    bufset = [
        pltpu.VMEM((B,), jnp.int32),
        pltpu.VMEM((B,), jnp.int32),
        pltpu.VMEM((B, H), jnp.float32),
        pltpu.VMEM((B, H), jnp.float32),
        pltpu.VMEM((B, H), jnp.float32),
        pltpu.VMEM((B, H), jnp.float32),
    ]
    f = pl.kernel(
        _sc_edge_body,
        out_type=jax.ShapeDtypeStruct((2 * NPAD, H), jnp.float32),
        mesh=mesh,
        scratch_types=bufset + bufset + [
            pltpu.VMEM_SHARED((NPAD, H), jnp.float32),
            pltpu.SemaphoreType.DMA,
            pltpu.SemaphoreType.DMA,
        ],
    )
    return f(k_tab, q_tab, v_tab, e_tab, src, dst)


def _sc_pair_body(ym_hbm, yv_hbm, i0_hbm, i1_hbm, out_hbm,
                  i0_v, i1_v, mr, vr, sem):
    c = lax.axis_index("c")
    s = lax.axis_index("s")
    base0 = (c * 16 + s) * (LP // NTILES)

    def step(g, carry):
        base = pl.multiple_of(base0 + g * B, 8)
        pltpu.sync_copy(i0_hbm.at[pl.ds(base, B)], i0_v)
        pltpu.sync_copy(i1_hbm.at[pl.ds(base, B)], i1_v)
        h1 = pltpu.async_copy(ym_hbm.at[i0_v], mr, sem)
        h2 = pltpu.async_copy(yv_hbm.at[i1_v], vr, sem)
        h1.wait()
        h2.wait()

        def row(i, carry2):
            mr[i, pl.ds(0, 16)] = mr[i, pl.ds(0, 16)] + vr[i, pl.ds(0, 16)]
            return carry2
        lax.fori_loop(0, B, row, 0)
        pltpu.sync_copy(mr, out_hbm.at[pl.ds(base, B)])
        return carry
    lax.fori_loop(0, (LP // NTILES) // B, step, 0)


def _sc_pairsum(ym, yv, i0, i1):
    mesh = plsc.VectorSubcoreMesh(core_axis_name="c", subcore_axis_name="s")
    f = pl.kernel(
        _sc_pair_body,
        out_type=jax.ShapeDtypeStruct((LP, PADC), jnp.float32),
        mesh=mesh,
        scratch_types=[
            pltpu.VMEM((B,), jnp.int32),
            pltpu.VMEM((B,), jnp.int32),
            pltpu.VMEM((B, PADC), jnp.float32),
            pltpu.VMEM((B, PADC), jnp.float32),
            pltpu.SemaphoreType.DMA,
        ],
    )
    return f(ym, yv, i0, i1)


def kernel(x_mouse, x_virus, edge_attr_mv, edge_attr_vm, params,
           edge_index_mv, edge_index_vm, edge_label_index):
    p = params

    def catw(*names):
        return jnp.concatenate([p[n] for n in names], axis=1)

    def catb(*names):
        return jnp.concatenate([p[n] for n in names])

    wm_cat = catw('l1_mv_Wq', 'l1_mv_Wv', 'l1_vm_Wk', 'l1_vm_Ws')
    bm_cat = catb('l1_mv_bq', 'l1_mv_bv', 'l1_vm_bk', 'l1_vm_bs')
    wm1 = p['mouse_lin_W'] @ wm_cat
    bm1 = p['mouse_lin_b'] @ wm_cat + bm_cat
    wv_cat = catw('l1_mv_Wk', 'l1_mv_Ws', 'l1_vm_Wq', 'l1_vm_Wv')
    bv_cat = catb('l1_mv_bk', 'l1_mv_bs', 'l1_vm_bq', 'l1_vm_bv')
    wv1 = p['virus_lin_W'] @ wv_cat
    bv1 = p['virus_lin_b'] @ wv_cat + bv_cat
    wm2 = catw('l2_mv_Wq', 'l2_mv_Wv', 'l2_vm_Wk', 'l2_vm_Ws')
    bm2 = catb('l2_mv_bq', 'l2_mv_bv', 'l2_vm_bk', 'l2_vm_bs')
    wv2 = catw('l2_mv_Wk', 'l2_mv_Ws', 'l2_vm_Wq', 'l2_vm_Wv')
    bv2 = catb('l2_mv_bk', 'l2_mv_bs', 'l2_vm_bq', 'l2_vm_bv')
    wtop = jnp.zeros((H, PADC), jnp.float32).at[:, :2].set(p['cls_W'][:H])
    wbot = jnp.zeros((H, PADC), jnp.float32).at[:, :2].set(p['cls_W'][H:])
    btop = jnp.zeros((PADC,), jnp.float32).at[:2].set(p['cls_b'])
    bzero = jnp.zeros((PADC,), jnp.float32)

    smv = edge_index_mv[0]
    dmv = edge_index_mv[1]
    svm = edge_index_vm[0]
    dvm = edge_index_vm[1]
    i0 = jnp.pad(edge_label_index[0], (0, LP - L))
    i1 = jnp.pad(edge_label_index[1], (0, LP - L))

    e1mv, = _mm_split(edge_attr_mv, p['l1_mv_We'], p['l1_mv_be'], 1, 4000)
    e1vm, = _mm_split(edge_attr_vm, p['l1_vm_We'], p['l1_vm_be'], 1, 4000)
    e2mv, = _mm_split(edge_attr_mv, p['l2_mv_We'], p['l2_mv_be'], 1, 4000)
    e2vm, = _mm_split(edge_attr_vm, p['l2_vm_We'], p['l2_vm_be'], 1, 4000)

    q1mv, v1mv, k1vm, s1vm = _mm_split(x_mouse, wm1, bm1, 4, 1000)
    k1mv, s1mv, q1vm, v1vm = _mm_split(x_virus, wv1, bv1, 4, 1000)

    agg1mv = _sc_edge(k1mv, q1mv, v1mv, e1mv, smv, dmv)
    agg1vm = _sc_edge(k1vm, q1vm, v1vm, e1vm, svm, dvm)

    q2mv, v2mv, k2vm, s2vm = _mm_combine(agg1vm, s1vm, wm2, bm2, 4)
    k2mv, s2mv, q2vm, v2vm = _mm_combine(agg1mv, s1mv, wv2, bv2, 4)

    agg2mv = _sc_edge(k2mv, q2mv, v2mv, e2mv, smv, dmv)
    agg2vm = _sc_edge(k2vm, q2vm, v2vm, e2vm, svm, dvm)

    ym, = _mm_combine(agg2vm, s2vm, wtop, btop, 1)
    yv, = _mm_combine(agg2mv, s2mv, wbot, bzero, 1)

    out = _sc_pairsum(ym, yv, i0, i1)
    return out[:L, :2]


# speedup vs baseline: 1.2877x; 1.0175x over previous
import functools

import jax
import jax.numpy as jnp
from jax import lax
from jax.experimental import pallas as pl
from jax.experimental.pallas import tpu as pltpu
from jax.experimental.pallas import tpu_sc as plsc

H = 128
D_E = 16
N = 10000
E = 320000
L = 50000
LP = 51200
NTILES = 32
EPT = E // NTILES
B = 40
BP = 80
STEPS = EPT // B
CH = 10
NPAD = 10240
SR = NPAD // 16
ZR = 40
PADC = 128
LPT = LP // NTILES


def _mm_body(x_ref, w_ref, b_ref, *out_refs):
    acc = jnp.dot(x_ref[...], w_ref[...], preferred_element_type=jnp.float32)
    acc = acc + b_ref[...]
    for j, o in enumerate(out_refs):
        o[...] = acc[:, j * H:(j + 1) * H]


def _mm_split(x, w, b, nout, rows):
    n = x.shape[0]
    grid = n // rows
    kdim = x.shape[1]
    return pl.pallas_call(
        _mm_body,
        grid=(grid,),
        in_specs=[
            pl.BlockSpec((rows, kdim), lambda i: (i, 0)),
            pl.BlockSpec((kdim, w.shape[1]), lambda i: (0, 0)),
            pl.BlockSpec((1, w.shape[1]), lambda i: (0, 0)),
        ],
        out_specs=[pl.BlockSpec((rows, H), lambda i: (i, 0))
                   for _ in range(nout)],
        out_shape=[jax.ShapeDtypeStruct((n, H), jnp.float32)
                   for _ in range(nout)],
    )(x, w, b.reshape(1, -1))


def _combine_body(a0_ref, a1_ref, s_ref, w_ref, b_ref, *out_refs):
    h = jnp.maximum(a0_ref[...] + a1_ref[...] + s_ref[...], 0.0)
    acc = jnp.dot(h, w_ref[...], preferred_element_type=jnp.float32)
    acc = acc + b_ref[...]
    wout = w_ref.shape[1] // len(out_refs)
    for j, o in enumerate(out_refs):
        o[...] = acc[:, j * wout:(j + 1) * wout]


def _mm_combine(aggflat, s, w, b, nout, rows=80):
    grid = N // rows
    nblk = NPAD // rows
    wout = w.shape[1] // nout
    return pl.pallas_call(
        _combine_body,
        grid=(grid,),
        in_specs=[
            pl.BlockSpec((rows, H), lambda i: (i, 0)),
            pl.BlockSpec((rows, H), lambda i, nblk=nblk: (i + nblk, 0)),
            pl.BlockSpec((rows, H), lambda i: (i, 0)),
            pl.BlockSpec((H, w.shape[1]), lambda i: (0, 0)),
            pl.BlockSpec((1, w.shape[1]), lambda i: (0, 0)),
        ],
        out_specs=[pl.BlockSpec((rows, wout), lambda i: (i, 0))
                   for _ in range(nout)],
        out_shape=[jax.ShapeDtypeStruct((N, wout), jnp.float32)
                   for _ in range(nout)],
    )(aggflat, aggflat, s, w, b.reshape(1, -1))


def _sc_edge_body(k_hbm, q_hbm, v_hbm, e_hbm, src_hbm, dst_hbm, out_hbm,
                  src0, dst0, kr0, qr0, vr0, er0,
                  src1, dst1, kr1, qr1, vr1, er1, agg_sh, sem0, sem1):
    c = lax.axis_index("c")
    s = lax.axis_index("s")

    def zrow(i, carry):
        for j in range(H // 16):
            kr0[i, pl.ds(j * 16, 16)] = jnp.zeros((16,), jnp.float32)
        return carry
    lax.fori_loop(0, ZR, zrow, 0)
    for m in range(SR // ZR):
        off = pl.multiple_of(s * SR + m * ZR, 8)
        pltpu.sync_copy(kr0, agg_sh.at[pl.ds(off, ZR)])
    plsc.subcore_barrier()

    base0 = (c * 16 + s) * EPT
    sets = ((src0, dst0, kr0, qr0, vr0, er0, sem0),
            (src1, dst1, kr1, qr1, vr1, er1, sem1))

    def issue(g, st):
        sv, dv, kr, qr, vr, er, sem = st
        base = pl.multiple_of(base0 + g * B, 8)
        pltpu.sync_copy(src_hbm.at[pl.ds(base, B)], sv)
        pltpu.sync_copy(dst_hbm.at[pl.ds(base, B)], dv)
        return (pltpu.async_copy(k_hbm.at[dv], kr, sem),
                pltpu.async_copy(q_hbm.at[sv], qr, sem),
                pltpu.async_copy(v_hbm.at[sv], vr, sem),
                pltpu.async_copy(e_hbm.at[pl.ds(base, B)], er, sem))

    def compute(st, hs):
        sv, dv, kr, qr, vr, er, sem = st
        for h in hs:
            h.wait()

        def row(i, carry2):
            for j in range(H // 16):
                sl = pl.ds(j * 16, 16)
                x = kr[i, sl] + qr[i, sl] + er[i, sl]
                gate = 1.0 / (1.0 + jnp.exp(-x))
                kr[i, sl] = gate * vr[i, sl]
            return carry2
        lax.fori_loop(0, B, row, 0)
        pltpu.sync_copy(kr, agg_sh.at[dv], add=True)

    def chunk(cc, carry):
        g0 = cc * CH
        hs = issue(g0, sets[0])
        for u in range(CH):
            nh = issue(g0 + u + 1, sets[(u + 1) % 2]) if u + 1 < CH else None
            compute(sets[u % 2], hs)
            hs = nh
        return carry
    lax.fori_loop(0, STEPS // CH, chunk, 0)
    plsc.subcore_barrier()

    for m in range(SR // ZR):
        off = pl.multiple_of(s * SR + m * ZR, 8)
        pltpu.sync_copy(agg_sh.at[pl.ds(off, ZR)],
                        out_hbm.at[pl.ds(pl.multiple_of(c * NPAD + off, 8),
                                         ZR)])


def _sc_edge(k_tab, q_tab, v_tab, e_tab, src, dst):
    mesh = plsc.VectorSubcoreMesh(core_axis_name="c", subcore_axis_name="s")
    bufset = [
        pltpu.VMEM((B,), jnp.int32),
        pltpu.VMEM((B,), jnp.int32),
        pltpu.VMEM((B, H), jnp.float32),
        pltpu.VMEM((B, H), jnp.float32),
        pltpu.VMEM((B, H), jnp.float32),
        pltpu.VMEM((B, H), jnp.float32),
    ]
    f = pl.kernel(
        _sc_edge_body,
        out_type=jax.ShapeDtypeStruct((2 * NPAD, H), jnp.float32),
        mesh=mesh,
        scratch_types=bufset + bufset + [
            pltpu.VMEM_SHARED((NPAD, H), jnp.float32),
            pltpu.SemaphoreType.DMA,
            pltpu.SemaphoreType.DMA,
        ],
    )
    return f(k_tab, q_tab, v_tab, e_tab, src, dst)


def _sc_pair_body(ym_hbm, yv_hbm, i0_hbm, i1_hbm, out_hbm,
                  i0a, i1a, mra, vra, i0b, i1b, mrb, vrb, sema, semb):
    c = lax.axis_index("c")
    s = lax.axis_index("s")
    base0 = (c * 16 + s) * LPT
    sets = ((i0a, i1a, mra, vra, sema), (i0b, i1b, mrb, vrb, semb))

    def issue(g, st):
        i0v, i1v, mr, vr, sem = st
        base = pl.multiple_of(base0 + g * BP, 8)
        pltpu.sync_copy(i0_hbm.at[pl.ds(base, BP)], i0v)
        pltpu.sync_copy(i1_hbm.at[pl.ds(base, BP)], i1v)
        return (pltpu.async_copy(ym_hbm.at[i0v], mr, sem),
                pltpu.async_copy(yv_hbm.at[i1v], vr, sem))

    def compute(g, st, hs):
        i0v, i1v, mr, vr, sem = st
        for h in hs:
            h.wait()

        def row(i, carry2):
            mr[i, pl.ds(0, 16)] = mr[i, pl.ds(0, 16)] + vr[i, pl.ds(0, 16)]
            return carry2
        lax.fori_loop(0, BP, row, 0)
        base = pl.multiple_of(base0 + g * BP, 8)
        pltpu.sync_copy(mr, out_hbm.at[pl.ds(base, BP)])

    def chunk(cc, carry):
        g0 = cc * CH
        hs = issue(g0, sets[0])
        for u in range(CH):
            nh = issue(g0 + u + 1, sets[(u + 1) % 2]) if u + 1 < CH else None
            compute(g0 + u, sets[u % 2], hs)
            hs = nh
        return carry
    lax.fori_loop(0, (LPT // BP) // CH, chunk, 0)


def _sc_pairsum(ym, yv, i0, i1):
    mesh = plsc.VectorSubcoreMesh(core_axis_name="c", subcore_axis_name="s")
    bufset = [
        pltpu.VMEM((BP,), jnp.int32),
        pltpu.VMEM((BP,), jnp.int32),
        pltpu.VMEM((BP, PADC), jnp.float32),
        pltpu.VMEM((BP, PADC), jnp.float32),
    ]
    f = pl.kernel(
        _sc_pair_body,
        out_type=jax.ShapeDtypeStruct((LP, PADC), jnp.float32),
        mesh=mesh,
        scratch_types=bufset + bufset + [
            pltpu.SemaphoreType.DMA,
            pltpu.SemaphoreType.DMA,
        ],
    )
    return f(ym, yv, i0, i1)


def kernel(x_mouse, x_virus, edge_attr_mv, edge_attr_vm, params,
           edge_index_mv, edge_index_vm, edge_label_index):
    p = params

    def catw(*names):
        return jnp.concatenate([p[n] for n in names], axis=1)

    def catb(*names):
        return jnp.concatenate([p[n] for n in names])

    wm_cat = catw('l1_mv_Wq', 'l1_mv_Wv', 'l1_vm_Wk', 'l1_vm_Ws')
    bm_cat = catb('l1_mv_bq', 'l1_mv_bv', 'l1_vm_bk', 'l1_vm_bs')
    wm1 = p['mouse_lin_W'] @ wm_cat
    bm1 = p['mouse_lin_b'] @ wm_cat + bm_cat
    wv_cat = catw('l1_mv_Wk', 'l1_mv_Ws', 'l1_vm_Wq', 'l1_vm_Wv')
    bv_cat = catb('l1_mv_bk', 'l1_mv_bs', 'l1_vm_bq', 'l1_vm_bv')
    wv1 = p['virus_lin_W'] @ wv_cat
    bv1 = p['virus_lin_b'] @ wv_cat + bv_cat
    wm2 = catw('l2_mv_Wq', 'l2_mv_Wv', 'l2_vm_Wk', 'l2_vm_Ws')
    bm2 = catb('l2_mv_bq', 'l2_mv_bv', 'l2_vm_bk', 'l2_vm_bs')
    wv2 = catw('l2_mv_Wk', 'l2_mv_Ws', 'l2_vm_Wq', 'l2_vm_Wv')
    bv2 = catb('l2_mv_bk', 'l2_mv_bs', 'l2_vm_bq', 'l2_vm_bv')
    wtop = jnp.zeros((H, PADC), jnp.float32).at[:, :2].set(p['cls_W'][:H])
    wbot = jnp.zeros((H, PADC), jnp.float32).at[:, :2].set(p['cls_W'][H:])
    btop = jnp.zeros((PADC,), jnp.float32).at[:2].set(p['cls_b'])
    bzero = jnp.zeros((PADC,), jnp.float32)

    smv = edge_index_mv[0]
    dmv = edge_index_mv[1]
    svm = edge_index_vm[0]
    dvm = edge_index_vm[1]
    i0 = jnp.pad(edge_label_index[0], (0, LP - L))
    i1 = jnp.pad(edge_label_index[1], (0, LP - L))

    e1mv, = _mm_split(edge_attr_mv, p['l1_mv_We'], p['l1_mv_be'], 1, 4000)
    e1vm, = _mm_split(edge_attr_vm, p['l1_vm_We'], p['l1_vm_be'], 1, 4000)
    e2mv, = _mm_split(edge_attr_mv, p['l2_mv_We'], p['l2_mv_be'], 1, 4000)
    e2vm, = _mm_split(edge_attr_vm, p['l2_vm_We'], p['l2_vm_be'], 1, 4000)

    q1mv, v1mv, k1vm, s1vm = _mm_split(x_mouse, wm1, bm1, 4, 1000)
    k1mv, s1mv, q1vm, v1vm = _mm_split(x_virus, wv1, bv1, 4, 1000)

    agg1mv = _sc_edge(k1mv, q1mv, v1mv, e1mv, smv, dmv)
    agg1vm = _sc_edge(k1vm, q1vm, v1vm, e1vm, svm, dvm)

    q2mv, v2mv, k2vm, s2vm = _mm_combine(agg1vm, s1vm, wm2, bm2, 4)
    k2mv, s2mv, q2vm, v2vm = _mm_combine(agg1mv, s1mv, wv2, bv2, 4)

    agg2mv = _sc_edge(k2mv, q2mv, v2mv, e2mv, smv, dmv)
    agg2vm = _sc_edge(k2vm, q2vm, v2vm, e2vm, svm, dvm)

    ym, = _mm_combine(agg2vm, s2vm, wtop, btop, 1)
    yv, = _mm_combine(agg2mv, s2mv, wbot, bzero, 1)

    out = _sc_pairsum(ym, yv, i0, i1)
    return out[:L, :2]


# speedup vs baseline: 1.4563x; 1.1310x over previous
import functools

import jax
import jax.numpy as jnp
from jax import lax
from jax.experimental import pallas as pl
from jax.experimental.pallas import tpu as pltpu
from jax.experimental.pallas import tpu_sc as plsc

H = 128
D_E = 16
N = 10000
E = 320000
L = 50000
LP = 51200
NTILES = 32
EPT = E // NTILES
B = 40
BP = 80
STEPS = EPT // B
CH = 25
CHP = 10
NPAD = 10240
SR = NPAD // 16
ZR = 40
PADC = 128
LPT = LP // NTILES


def _mm_body(x_ref, w_ref, b_ref, *out_refs):
    acc = jnp.dot(x_ref[...], w_ref[...], preferred_element_type=jnp.float32)
    acc = acc + b_ref[...]
    for j, o in enumerate(out_refs):
        o[...] = acc[:, j * H:(j + 1) * H]


def _mm_split(x, w, b, nout, rows):
    n = x.shape[0]
    grid = n // rows
    kdim = x.shape[1]
    return pl.pallas_call(
        _mm_body,
        grid=(grid,),
        in_specs=[
            pl.BlockSpec((rows, kdim), lambda i: (i, 0)),
            pl.BlockSpec((kdim, w.shape[1]), lambda i: (0, 0)),
            pl.BlockSpec((1, w.shape[1]), lambda i: (0, 0)),
        ],
        out_specs=[pl.BlockSpec((rows, H), lambda i: (i, 0))
                   for _ in range(nout)],
        out_shape=[jax.ShapeDtypeStruct((n, H), jnp.float32)
                   for _ in range(nout)],
    )(x, w, b.reshape(1, -1))


def _combine_body(a0_ref, a1_ref, s_ref, w_ref, b_ref, *out_refs):
    h = jnp.maximum(a0_ref[...] + a1_ref[...] + s_ref[...], 0.0)
    acc = jnp.dot(h, w_ref[...], preferred_element_type=jnp.float32)
    acc = acc + b_ref[...]
    wout = w_ref.shape[1] // len(out_refs)
    for j, o in enumerate(out_refs):
        o[...] = acc[:, j * wout:(j + 1) * wout]


def _mm_combine(aggflat, s, w, b, nout, rows=80):
    grid = N // rows
    nblk = NPAD // rows
    wout = w.shape[1] // nout
    return pl.pallas_call(
        _combine_body,
        grid=(grid,),
        in_specs=[
            pl.BlockSpec((rows, H), lambda i: (i, 0)),
            pl.BlockSpec((rows, H), lambda i, nblk=nblk: (i + nblk, 0)),
            pl.BlockSpec((rows, H), lambda i: (i, 0)),
            pl.BlockSpec((H, w.shape[1]), lambda i: (0, 0)),
            pl.BlockSpec((1, w.shape[1]), lambda i: (0, 0)),
        ],
        out_specs=[pl.BlockSpec((rows, wout), lambda i: (i, 0))
                   for _ in range(nout)],
        out_shape=[jax.ShapeDtypeStruct((N, wout), jnp.float32)
                   for _ in range(nout)],
    )(aggflat, aggflat, s, w, b.reshape(1, -1))


def _sc_edge_body(k_hbm, q_hbm, v_hbm, e_hbm, src_hbm, dst_hbm, out_hbm,
                  src0, dst0, kr0, qr0, vr0, er0,
                  src1, dst1, kr1, qr1, vr1, er1, agg_sh,
                  sem0, sem1, semi0, semi1):
    c = lax.axis_index("c")
    s = lax.axis_index("s")

    def zrow(i, carry):
        for j in range(H // 16):
            kr0[i, pl.ds(j * 16, 16)] = jnp.zeros((16,), jnp.float32)
        return carry
    lax.fori_loop(0, ZR, zrow, 0)
    for m in range(SR // ZR):
        off = pl.multiple_of(s * SR + m * ZR, 8)
        pltpu.sync_copy(kr0, agg_sh.at[pl.ds(off, ZR)])
    plsc.subcore_barrier()

    base0 = (c * 16 + s) * EPT
    sets = ((src0, dst0, kr0, qr0, vr0, er0, sem0, semi0),
            (src1, dst1, kr1, qr1, vr1, er1, sem1, semi1))

    def issue_idx(g, st):
        sv, dv, kr, qr, vr, er, sem, semi = st
        base = pl.multiple_of(base0 + g * B, 8)
        return (pltpu.async_copy(src_hbm.at[pl.ds(base, B)], sv, semi),
                pltpu.async_copy(dst_hbm.at[pl.ds(base, B)], dv, semi))

    def issue_gather(g, st, ih):
        sv, dv, kr, qr, vr, er, sem, semi = st
        for h in ih:
            h.wait()
        base = pl.multiple_of(base0 + g * B, 8)
        return (pltpu.async_copy(k_hbm.at[dv], kr, sem),
                pltpu.async_copy(q_hbm.at[sv], qr, sem),
                pltpu.async_copy(v_hbm.at[sv], vr, sem),
                pltpu.async_copy(e_hbm.at[pl.ds(base, B)], er, sem))

    def compute(st, hs):
        sv, dv, kr, qr, vr, er, sem, semi = st
        for h in hs:
            h.wait()

        def row(i, carry2):
            for j in range(H // 16):
                sl = pl.ds(j * 16, 16)
                x = kr[i, sl] + qr[i, sl] + er[i, sl]
                gate = 1.0 / (1.0 + jnp.exp(-x))
                kr[i, sl] = gate * vr[i, sl]
            return carry2
        lax.fori_loop(0, B, row, 0)
        pltpu.sync_copy(kr, agg_sh.at[dv], add=True)

    def chunk(cc, carry):
        g0 = cc * CH
        ih = issue_idx(g0, sets[0])
        ihn = issue_idx(g0 + 1, sets[1])
        hs = issue_gather(g0, sets[0], ih)
        for u in range(CH):
            nh = (issue_gather(g0 + u + 1, sets[(u + 1) % 2], ihn)
                  if u + 1 < CH else None)
            compute(sets[u % 2], hs)
            ihn = issue_idx(g0 + u + 2, sets[u % 2]) if u + 2 < CH else None
            hs = nh
        return carry
    lax.fori_loop(0, STEPS // CH, chunk, 0)
    plsc.subcore_barrier()

    for m in range(SR // ZR):
        off = pl.multiple_of(s * SR + m * ZR, 8)
        pltpu.sync_copy(agg_sh.at[pl.ds(off, ZR)],
                        out_hbm.at[pl.ds(pl.multiple_of(c * NPAD + off, 8),
                                         ZR)])


def _sc_edge(k_tab, q_tab, v_tab, e_tab, src, dst):
    mesh = plsc.VectorSubcoreMesh(core_axis_name="c", subcore_axis_name="s")
    bufset = [
        pltpu.VMEM((B,), jnp.int32),
        pltpu.VMEM((B,), jnp.int32),
        pltpu.VMEM((B, H), jnp.float32),
        pltpu.VMEM((B, H), jnp.float32),
        pltpu.VMEM((B, H), jnp.float32),
        pltpu.VMEM((B, H), jnp.float32),
    ]
    f = pl.kernel(
        _sc_edge_body,
        out_type=jax.ShapeDtypeStruct((2 * NPAD, H), jnp.float32),
        mesh=mesh,
        scratch_types=bufset + bufset + [
            pltpu.VMEM_SHARED((NPAD, H), jnp.float32),
            pltpu.SemaphoreType.DMA,
            pltpu.SemaphoreType.DMA,
            pltpu.SemaphoreType.DMA,
            pltpu.SemaphoreType.DMA,
        ],
    )
    return f(k_tab, q_tab, v_tab, e_tab, src, dst)


def _sc_pair_body(ym_hbm, yv_hbm, i0_hbm, i1_hbm, out_hbm,
                  i0a, i1a, mra, vra, i0b, i1b, mrb, vrb, sema, semb):
    c = lax.axis_index("c")
    s = lax.axis_index("s")
    base0 = (c * 16 + s) * LPT
    sets = ((i0a, i1a, mra, vra, sema), (i0b, i1b, mrb, vrb, semb))

    def issue(g, st):
        i0v, i1v, mr, vr, sem = st
        base = pl.multiple_of(base0 + g * BP, 8)
        pltpu.sync_copy(i0_hbm.at[pl.ds(base, BP)], i0v)
        pltpu.sync_copy(i1_hbm.at[pl.ds(base, BP)], i1v)
        return (pltpu.async_copy(ym_hbm.at[i0v], mr, sem),
                pltpu.async_copy(yv_hbm.at[i1v], vr, sem))

    def compute(g, st, hs):
        i0v, i1v, mr, vr, sem = st
        for h in hs:
            h.wait()

        def row(i, carry2):
            mr[i, pl.ds(0, 16)] = mr[i, pl.ds(0, 16)] + vr[i, pl.ds(0, 16)]
            return carry2
        lax.fori_loop(0, BP, row, 0)
        base = pl.multiple_of(base0 + g * BP, 8)
        pltpu.sync_copy(mr, out_hbm.at[pl.ds(base, BP)])

    def chunk(cc, carry):
        g0 = cc * CHP
        hs = issue(g0, sets[0])
        for u in range(CHP):
            nh = issue(g0 + u + 1, sets[(u + 1) % 2]) if u + 1 < CHP else None
            compute(g0 + u, sets[u % 2], hs)
            hs = nh
        return carry
    lax.fori_loop(0, (LPT // BP) // CHP, chunk, 0)


def _sc_pairsum(ym, yv, i0, i1):
    mesh = plsc.VectorSubcoreMesh(core_axis_name="c", subcore_axis_name="s")
    bufset = [
        pltpu.VMEM((BP,), jnp.int32),
        pltpu.VMEM((BP,), jnp.int32),
        pltpu.VMEM((BP, PADC), jnp.float32),
        pltpu.VMEM((BP, PADC), jnp.float32),
    ]
    f = pl.kernel(
        _sc_pair_body,
        out_type=jax.ShapeDtypeStruct((LP, PADC), jnp.float32),
        mesh=mesh,
        scratch_types=bufset + bufset + [
            pltpu.SemaphoreType.DMA,
            pltpu.SemaphoreType.DMA,
        ],
    )
    return f(ym, yv, i0, i1)


def kernel(x_mouse, x_virus, edge_attr_mv, edge_attr_vm, params,
           edge_index_mv, edge_index_vm, edge_label_index):
    p = params

    def catw(*names):
        return jnp.concatenate([p[n] for n in names], axis=1)

    def catb(*names):
        return jnp.concatenate([p[n] for n in names])

    wm_cat = catw('l1_mv_Wq', 'l1_mv_Wv', 'l1_vm_Wk', 'l1_vm_Ws')
    bm_cat = catb('l1_mv_bq', 'l1_mv_bv', 'l1_vm_bk', 'l1_vm_bs')
    wm1 = p['mouse_lin_W'] @ wm_cat
    bm1 = p['mouse_lin_b'] @ wm_cat + bm_cat
    wv_cat = catw('l1_mv_Wk', 'l1_mv_Ws', 'l1_vm_Wq', 'l1_vm_Wv')
    bv_cat = catb('l1_mv_bk', 'l1_mv_bs', 'l1_vm_bq', 'l1_vm_bv')
    wv1 = p['virus_lin_W'] @ wv_cat
    bv1 = p['virus_lin_b'] @ wv_cat + bv_cat
    wm2 = catw('l2_mv_Wq', 'l2_mv_Wv', 'l2_vm_Wk', 'l2_vm_Ws')
    bm2 = catb('l2_mv_bq', 'l2_mv_bv', 'l2_vm_bk', 'l2_vm_bs')
    wv2 = catw('l2_mv_Wk', 'l2_mv_Ws', 'l2_vm_Wq', 'l2_vm_Wv')
    bv2 = catb('l2_mv_bk', 'l2_mv_bs', 'l2_vm_bq', 'l2_vm_bv')
    wtop = jnp.zeros((H, PADC), jnp.float32).at[:, :2].set(p['cls_W'][:H])
    wbot = jnp.zeros((H, PADC), jnp.float32).at[:, :2].set(p['cls_W'][H:])
    btop = jnp.zeros((PADC,), jnp.float32).at[:2].set(p['cls_b'])
    bzero = jnp.zeros((PADC,), jnp.float32)

    smv = edge_index_mv[0]
    dmv = edge_index_mv[1]
    svm = edge_index_vm[0]
    dvm = edge_index_vm[1]
    i0 = jnp.pad(edge_label_index[0], (0, LP - L))
    i1 = jnp.pad(edge_label_index[1], (0, LP - L))

    e1mv, = _mm_split(edge_attr_mv, p['l1_mv_We'], p['l1_mv_be'], 1, 4000)
    e1vm, = _mm_split(edge_attr_vm, p['l1_vm_We'], p['l1_vm_be'], 1, 4000)
    e2mv, = _mm_split(edge_attr_mv, p['l2_mv_We'], p['l2_mv_be'], 1, 4000)
    e2vm, = _mm_split(edge_attr_vm, p['l2_vm_We'], p['l2_vm_be'], 1, 4000)

    q1mv, v1mv, k1vm, s1vm = _mm_split(x_mouse, wm1, bm1, 4, 1000)
    k1mv, s1mv, q1vm, v1vm = _mm_split(x_virus, wv1, bv1, 4, 1000)

    agg1mv = _sc_edge(k1mv, q1mv, v1mv, e1mv, smv, dmv)
    agg1vm = _sc_edge(k1vm, q1vm, v1vm, e1vm, svm, dvm)

    q2mv, v2mv, k2vm, s2vm = _mm_combine(agg1vm, s1vm, wm2, bm2, 4)
    k2mv, s2mv, q2vm, v2vm = _mm_combine(agg1mv, s1mv, wv2, bv2, 4)

    agg2mv = _sc_edge(k2mv, q2mv, v2mv, e2mv, smv, dmv)
    agg2vm = _sc_edge(k2vm, q2vm, v2vm, e2vm, svm, dvm)

    ym, = _mm_combine(agg2vm, s2vm, wtop, btop, 1)
    yv, = _mm_combine(agg2mv, s2mv, wbot, bzero, 1)

    out = _sc_pairsum(ym, yv, i0, i1)
    return out[:L, :2]


# speedup vs baseline: 1.5346x; 1.0538x over previous
import functools

import jax
import jax.numpy as jnp
from jax import lax
from jax.experimental import pallas as pl
from jax.experimental.pallas import tpu as pltpu
from jax.experimental.pallas import tpu_sc as plsc

H = 128
D_E = 16
N = 10000
E = 320000
L = 50000
LP = 51200
NTILES = 32
EPT = E // NTILES
B = 40
BP = 80
STEPS = EPT // B
CH = 25
CHP = 10
NPAD = 10240
SR = NPAD // 16
ZR = 40
PADC = 128
LPT = LP // NTILES


def _mm_body(x_ref, w_ref, b_ref, *out_refs):
    acc = jnp.dot(x_ref[...], w_ref[...], preferred_element_type=jnp.float32)
    acc = acc + b_ref[...]
    for j, o in enumerate(out_refs):
        o[...] = acc[:, j * H:(j + 1) * H]


def _mm_split(x, w, b, nout, rows):
    n = x.shape[0]
    grid = n // rows
    kdim = x.shape[1]
    return pl.pallas_call(
        _mm_body,
        grid=(grid,),
        in_specs=[
            pl.BlockSpec((rows, kdim), lambda i: (i, 0)),
            pl.BlockSpec((kdim, w.shape[1]), lambda i: (0, 0)),
            pl.BlockSpec((1, w.shape[1]), lambda i: (0, 0)),
        ],
        out_specs=[pl.BlockSpec((rows, H), lambda i: (i, 0))
                   for _ in range(nout)],
        out_shape=[jax.ShapeDtypeStruct((n, H), jnp.float32)
                   for _ in range(nout)],
    )(x, w, b.reshape(1, -1))


def _combine_body(a0_ref, a1_ref, s_ref, w_ref, b_ref, *out_refs):
    h = jnp.maximum(a0_ref[...] + a1_ref[...] + s_ref[...], 0.0)
    acc = jnp.dot(h, w_ref[...], preferred_element_type=jnp.float32)
    acc = acc + b_ref[...]
    wout = w_ref.shape[1] // len(out_refs)
    for j, o in enumerate(out_refs):
        o[...] = acc[:, j * wout:(j + 1) * wout]


def _mm_combine(aggflat, s, w, b, nout, rows=80):
    grid = N // rows
    nblk = NPAD // rows
    wout = w.shape[1] // nout
    return pl.pallas_call(
        _combine_body,
        grid=(grid,),
        in_specs=[
            pl.BlockSpec((rows, H), lambda i: (i, 0)),
            pl.BlockSpec((rows, H), lambda i, nblk=nblk: (i + nblk, 0)),
            pl.BlockSpec((rows, H), lambda i: (i, 0)),
            pl.BlockSpec((H, w.shape[1]), lambda i: (0, 0)),
            pl.BlockSpec((1, w.shape[1]), lambda i: (0, 0)),
        ],
        out_specs=[pl.BlockSpec((rows, wout), lambda i: (i, 0))
                   for _ in range(nout)],
        out_shape=[jax.ShapeDtypeStruct((N, wout), jnp.float32)
                   for _ in range(nout)],
    )(aggflat, aggflat, s, w, b.reshape(1, -1))


def _sc_edge_body(k_hbm, q_hbm, v_hbm, e_hbm, src_hbm, dst_hbm, out_hbm,
                  src0, dst0, src1, dst1, src2, dst2,
                  kr0, kr1, kr2, qr0, vr0, er0, qr1, vr1, er1, agg_sh,
                  semg0, semg1, semi0, semi1, sems0, sems1, sems2):
    c = lax.axis_index("c")
    s = lax.axis_index("s")

    def zrow(i, carry):
        for j in range(H // 16):
            kr0[i, pl.ds(j * 16, 16)] = jnp.zeros((16,), jnp.float32)
        return carry
    lax.fori_loop(0, ZR, zrow, 0)
    for m in range(SR // ZR):
        off = pl.multiple_of(s * SR + m * ZR, 8)
        pltpu.sync_copy(kr0, agg_sh.at[pl.ds(off, ZR)])
    plsc.subcore_barrier()

    base0 = (c * 16 + s) * EPT
    svs = (src0, src1, src2)
    dvs = (dst0, dst1, dst2)
    krs = (kr0, kr1, kr2)
    qrs = (qr0, qr1)
    vrs = (vr0, vr1)
    ers = (er0, er1)
    semg = (semg0, semg1)
    semi = (semi0, semi1)
    sems = (sems0, sems1, sems2)

    def issue_idx(g, u):
        base = pl.multiple_of(base0 + g * B, 8)
        return (pltpu.async_copy(src_hbm.at[pl.ds(base, B)], svs[u % 3],
                                 semi[u % 2]),
                pltpu.async_copy(dst_hbm.at[pl.ds(base, B)], dvs[u % 3],
                                 semi[u % 2]))

    def issue_gather(g, u, ih):
        for h in ih:
            h.wait()
        base = pl.multiple_of(base0 + g * B, 8)
        return (pltpu.async_copy(k_hbm.at[dvs[u % 3]], krs[u % 3],
                                 semg[u % 2]),
                pltpu.async_copy(q_hbm.at[svs[u % 3]], qrs[u % 2],
                                 semg[u % 2]),
                pltpu.async_copy(v_hbm.at[svs[u % 3]], vrs[u % 2],
                                 semg[u % 2]),
                pltpu.async_copy(e_hbm.at[pl.ds(base, B)], ers[u % 2],
                                 semg[u % 2]))

    def compute(u, hs):
        for h in hs:
            h.wait()
        kr, qr, vr, er = krs[u % 3], qrs[u % 2], vrs[u % 2], ers[u % 2]

        def row(i, carry2):
            for j in range(H // 16):
                sl = pl.ds(j * 16, 16)
                x = kr[i, sl] + qr[i, sl] + er[i, sl]
                gate = 1.0 / (1.0 + jnp.exp(-x))
                kr[i, sl] = gate * vr[i, sl]
            return carry2
        lax.fori_loop(0, B, row, 0)
        return pltpu.async_copy(kr, agg_sh.at[dvs[u % 3]], sems[u % 3],
                                add=True)

    def chunk(cc, carry):
        g0 = cc * CH
        sc = [None, None, None]
        ih = issue_idx(g0, 0)
        ihn = issue_idx(g0 + 1, 1)
        hs = issue_gather(g0, 0, ih)
        for u in range(CH):
            if u + 1 < CH:
                if sc[(u + 1) % 3] is not None:
                    sc[(u + 1) % 3].wait()
                    sc[(u + 1) % 3] = None
                nh = issue_gather(g0 + u + 1, u + 1, ihn)
            else:
                nh = None
            sc[u % 3] = compute(u, hs)
            if u + 2 < CH:
                if sc[(u + 2) % 3] is not None:
                    sc[(u + 2) % 3].wait()
                    sc[(u + 2) % 3] = None
                ihn = issue_idx(g0 + u + 2, u + 2)
            else:
                ihn = None
            hs = nh
        for h in sc:
            if h is not None:
                h.wait()
        return carry
    lax.fori_loop(0, STEPS // CH, chunk, 0)
    plsc.subcore_barrier()

    for m in range(SR // ZR):
        off = pl.multiple_of(s * SR + m * ZR, 8)
        pltpu.sync_copy(agg_sh.at[pl.ds(off, ZR)],
                        out_hbm.at[pl.ds(pl.multiple_of(c * NPAD + off, 8),
                                         ZR)])


def _sc_edge(k_tab, q_tab, v_tab, e_tab, src, dst):
    mesh = plsc.VectorSubcoreMesh(core_axis_name="c", subcore_axis_name="s")
    idxbufs = [pltpu.VMEM((B,), jnp.int32) for _ in range(6)]
    krbufs = [pltpu.VMEM((B, H), jnp.float32) for _ in range(3)]
    setbufs = [pltpu.VMEM((B, H), jnp.float32) for _ in range(6)]
    f = pl.kernel(
        _sc_edge_body,
        out_type=jax.ShapeDtypeStruct((2 * NPAD, H), jnp.float32),
        mesh=mesh,
        scratch_types=idxbufs + krbufs + setbufs + [
            pltpu.VMEM_SHARED((NPAD, H), jnp.float32),
        ] + [pltpu.SemaphoreType.DMA for _ in range(7)],
    )
    return f(k_tab, q_tab, v_tab, e_tab, src, dst)


def _sc_pair_body(ym_hbm, yv_hbm, i0_hbm, i1_hbm, out_hbm,
                  i0a, i1a, mra, vra, i0b, i1b, mrb, vrb, sema, semb):
    c = lax.axis_index("c")
    s = lax.axis_index("s")
    base0 = (c * 16 + s) * LPT
    sets = ((i0a, i1a, mra, vra, sema), (i0b, i1b, mrb, vrb, semb))

    def issue(g, st):
        i0v, i1v, mr, vr, sem = st
        base = pl.multiple_of(base0 + g * BP, 8)
        pltpu.sync_copy(i0_hbm.at[pl.ds(base, BP)], i0v)
        pltpu.sync_copy(i1_hbm.at[pl.ds(base, BP)], i1v)
        return (pltpu.async_copy(ym_hbm.at[i0v], mr, sem),
                pltpu.async_copy(yv_hbm.at[i1v], vr, sem))

    def compute(g, st, hs):
        i0v, i1v, mr, vr, sem = st
        for h in hs:
            h.wait()

        def row(i, carry2):
            mr[i, pl.ds(0, 16)] = mr[i, pl.ds(0, 16)] + vr[i, pl.ds(0, 16)]
            return carry2
        lax.fori_loop(0, BP, row, 0)
        base = pl.multiple_of(base0 + g * BP, 8)
        pltpu.sync_copy(mr, out_hbm.at[pl.ds(base, BP)])

    def chunk(cc, carry):
        g0 = cc * CHP
        hs = issue(g0, sets[0])
        for u in range(CHP):
            nh = issue(g0 + u + 1, sets[(u + 1) % 2]) if u + 1 < CHP else None
            compute(g0 + u, sets[u % 2], hs)
            hs = nh
        return carry
    lax.fori_loop(0, (LPT // BP) // CHP, chunk, 0)


def _sc_pairsum(ym, yv, i0, i1):
    mesh = plsc.VectorSubcoreMesh(core_axis_name="c", subcore_axis_name="s")
    bufset = [
        pltpu.VMEM((BP,), jnp.int32),
        pltpu.VMEM((BP,), jnp.int32),
        pltpu.VMEM((BP, PADC), jnp.float32),
        pltpu.VMEM((BP, PADC), jnp.float32),
    ]
    f = pl.kernel(
        _sc_pair_body,
        out_type=jax.ShapeDtypeStruct((LP, PADC), jnp.float32),
        mesh=mesh,
        scratch_types=bufset + bufset + [
            pltpu.SemaphoreType.DMA,
            pltpu.SemaphoreType.DMA,
        ],
    )
    return f(ym, yv, i0, i1)


def kernel(x_mouse, x_virus, edge_attr_mv, edge_attr_vm, params,
           edge_index_mv, edge_index_vm, edge_label_index):
    p = params

    def catw(*names):
        return jnp.concatenate([p[n] for n in names], axis=1)

    def catb(*names):
        return jnp.concatenate([p[n] for n in names])

    wm_cat = catw('l1_mv_Wq', 'l1_mv_Wv', 'l1_vm_Wk', 'l1_vm_Ws')
    bm_cat = catb('l1_mv_bq', 'l1_mv_bv', 'l1_vm_bk', 'l1_vm_bs')
    wm1 = p['mouse_lin_W'] @ wm_cat
    bm1 = p['mouse_lin_b'] @ wm_cat + bm_cat
    wv_cat = catw('l1_mv_Wk', 'l1_mv_Ws', 'l1_vm_Wq', 'l1_vm_Wv')
    bv_cat = catb('l1_mv_bk', 'l1_mv_bs', 'l1_vm_bq', 'l1_vm_bv')
    wv1 = p['virus_lin_W'] @ wv_cat
    bv1 = p['virus_lin_b'] @ wv_cat + bv_cat
    wm2 = catw('l2_mv_Wq', 'l2_mv_Wv', 'l2_vm_Wk', 'l2_vm_Ws')
    bm2 = catb('l2_mv_bq', 'l2_mv_bv', 'l2_vm_bk', 'l2_vm_bs')
    wv2 = catw('l2_mv_Wk', 'l2_mv_Ws', 'l2_vm_Wq', 'l2_vm_Wv')
    bv2 = catb('l2_mv_bk', 'l2_mv_bs', 'l2_vm_bq', 'l2_vm_bv')
    wtop = jnp.zeros((H, PADC), jnp.float32).at[:, :2].set(p['cls_W'][:H])
    wbot = jnp.zeros((H, PADC), jnp.float32).at[:, :2].set(p['cls_W'][H:])
    btop = jnp.zeros((PADC,), jnp.float32).at[:2].set(p['cls_b'])
    bzero = jnp.zeros((PADC,), jnp.float32)

    smv = edge_index_mv[0]
    dmv = edge_index_mv[1]
    svm = edge_index_vm[0]
    dvm = edge_index_vm[1]
    i0 = jnp.pad(edge_label_index[0], (0, LP - L))
    i1 = jnp.pad(edge_label_index[1], (0, LP - L))

    e1mv, = _mm_split(edge_attr_mv, p['l1_mv_We'], p['l1_mv_be'], 1, 4000)
    e1vm, = _mm_split(edge_attr_vm, p['l1_vm_We'], p['l1_vm_be'], 1, 4000)
    e2mv, = _mm_split(edge_attr_mv, p['l2_mv_We'], p['l2_mv_be'], 1, 4000)
    e2vm, = _mm_split(edge_attr_vm, p['l2_vm_We'], p['l2_vm_be'], 1, 4000)

    q1mv, v1mv, k1vm, s1vm = _mm_split(x_mouse, wm1, bm1, 4, 1000)
    k1mv, s1mv, q1vm, v1vm = _mm_split(x_virus, wv1, bv1, 4, 1000)

    agg1mv = _sc_edge(k1mv, q1mv, v1mv, e1mv, smv, dmv)
    agg1vm = _sc_edge(k1vm, q1vm, v1vm, e1vm, svm, dvm)

    q2mv, v2mv, k2vm, s2vm = _mm_combine(agg1vm, s1vm, wm2, bm2, 4)
    k2mv, s2mv, q2vm, v2vm = _mm_combine(agg1mv, s1mv, wv2, bv2, 4)

    agg2mv = _sc_edge(k2mv, q2mv, v2mv, e2mv, smv, dmv)
    agg2vm = _sc_edge(k2vm, q2vm, v2vm, e2vm, svm, dvm)

    ym, = _mm_combine(agg2vm, s2vm, wtop, btop, 1)
    yv, = _mm_combine(agg2mv, s2mv, wbot, bzero, 1)

    out = _sc_pairsum(ym, yv, i0, i1)
    return out[:L, :2]
